# baseline (speedup 1.0000x reference)
import functools

import jax
import jax.numpy as jnp
from jax import lax
from jax.experimental import pallas as pl
from jax.experimental.pallas import tpu as pltpu
from jax.experimental.pallas import tpu_sc as plsc

_NBUF = 2
_L = 16


def _soft_embedding_call(tokens, wte_pairs, learned_embedding, B, S, D, NT):
    NT8 = (NT // _L) * _L
    G = S - NT8
    info = plsc.get_sparse_core_info()
    NC, NS = info.num_cores, info.num_subcores
    NW = NC * NS
    RPW = B // NW
    C1 = min(G, 128)
    C2 = G - C1
    DP = 2 * D

    mesh = plsc.VectorSubcoreMesh(core_axis_name="c", subcore_axis_name="s")

    @functools.partial(
        pl.kernel,
        mesh=mesh,
        out_type=jax.ShapeDtypeStruct((B, S, DP), jnp.float32),
        compiler_params=pltpu.CompilerParams(use_tc_tiling_on_sc=True,
                                             needs_layout_passes=False),
        scratch_types=[
            pltpu.VMEM((RPW, S), jnp.int32),
            pltpu.VMEM((_NBUF, G), jnp.int32),
            pltpu.VMEM((_NBUF, G, DP), jnp.float32),
            pltpu.VMEM((_NBUF, S, DP), jnp.float32),
            pltpu.VMEM((NT * D,), jnp.float32),
            pltpu.SemaphoreType.DMA((_NBUF,)),
        ],
    )
    def soft_embed(tok_hbm, wte_hbm, le_hbm, out_hbm, toks_v, pidx_v, pair_v,
                   row_v, le_v, gsem):
        wid = lax.axis_index("s") * NC + lax.axis_index("c")
        base = wid * RPW
        pltpu.sync_copy(tok_hbm.at[pl.ds(base, RPW)], toks_v)
        pltpu.sync_copy(le_hbm, le_v)
        for b in range(_NBUF):
            for r in range(NT):
                for c in range(0, D, _L):
                    row_v[b, r, pl.ds(c, _L)] = le_v[pl.ds(r * D + c, _L)]

        iota = lax.iota(jnp.int32, _L)

        def make_pidx(g, b):
            for c in range(G // _L):
                t = toks_v[g, pl.ds(NT8 + c * _L, _L)]
                pidx_v[b, pl.ds(c * _L, _L)] = jax.lax.shift_right_logical(t, 1)
            if G % _L:
                t = toks_v[g, pl.ds(S - _L, _L)]
                pidx_v[b, pl.ds(G - _L, _L)] = jax.lax.shift_right_logical(t, 1)

        def gather_copies(g, b):
            cs = [pltpu.make_async_copy(
                wte_hbm.at[pidx_v.at[b, pl.ds(0, C1)]],
                pair_v.at[b, pl.ds(0, C1)], gsem.at[b])]
            if C2:
                cs.append(pltpu.make_async_copy(
                    wte_hbm.at[pidx_v.at[b, pl.ds(C1, C2)]],
                    pair_v.at[b, pl.ds(C1, C2)], gsem.at[b]))
            return cs

        def select_group(g, b, jj0):
            t = toks_v[g, pl.ds(NT8 + jj0, _L)]
            hvec = jnp.bitwise_and(t, 1) * D
            prows = iota + jj0
            srows = prows + NT8
            mask = srows >= NT
            for d in range(D):
                vals = plsc.load_gather(pair_v.at[b], [prows, hvec + d])
                plsc.store_scatter(row_v.at[b], [srows, iota * 0 + d], vals,
                                   mask=mask)

        def select(g, b):
            def grp(q, carry):
                select_group(g, b, q * _L)
                return carry
            lax.fori_loop(0, (G // _L), grp, 0)
            if G % _L:
                select_group(g, b, G - _L)

        def write_out(g, b):
            pltpu.sync_copy(row_v.at[b], out_hbm.at[base + g])

        def start(cs):
            for c in cs:
                c.start()

        def wait(cs):
            for c in cs:
                c.wait()

        for b in range(_NBUF):
            make_pidx(b, b)
            start(gather_copies(b, b))

        def outer(k, carry):
            for bb in range(_NBUF):
                g = k * _NBUF + bb
                wait(gather_copies(g, bb))
                select(g, bb)
                make_pidx(g + _NBUF, bb)
                start(gather_copies(g + _NBUF, bb))
                write_out(g, bb)
            return carry

        lax.fori_loop(0, (RPW - _NBUF) // _NBUF, outer, 0)

        for bb in range(_NBUF):
            g = RPW - _NBUF + bb
            wait(gather_copies(g, bb))
            select(g, bb)
            write_out(g, bb)

    return soft_embed(tokens, wte_pairs, learned_embedding)[:, :, :D]


def kernel(tokens, wte_weight, learned_embedding):
    B, S = tokens.shape
    V, D = wte_weight.shape
    NT = learned_embedding.shape[0]
    tokens = tokens.astype(jnp.int32)
    wte_pairs = wte_weight.reshape(V // 2, 2 * D)
    learned_flat = learned_embedding.reshape(-1)
    return _soft_embedding_call(
        tokens, wte_pairs, learned_flat, B, S, D, NT)


# speedup vs baseline: 1.4828x; 1.4828x over previous
import functools

import jax
import jax.numpy as jnp
from jax import lax
from jax.experimental import pallas as pl
from jax.experimental.pallas import tpu as pltpu
from jax.experimental.pallas import tpu_sc as plsc

_NBUF = 4


def _depad_tokens_call(tokens, B, S):
    info = plsc.get_sparse_core_info()
    NC, NS = info.num_cores, info.num_subcores
    RPW = B // (NC * NS)
    mesh = plsc.VectorSubcoreMesh(core_axis_name="c", subcore_axis_name="s")
    L = 16
    tail = S - L

    @functools.partial(
        pl.kernel,
        mesh=mesh,
        out_type=jax.ShapeDtypeStruct((B * S,), jnp.int32),
        compiler_params=pltpu.CompilerParams(use_tc_tiling_on_sc=True),
        scratch_types=[
            pltpu.VMEM((RPW, S), jnp.int32),
            pltpu.VMEM((RPW * S,), jnp.int32),
        ],
    )
    def depad(tok_hbm, out_hbm, stage_v, flat_v):
        wid = lax.axis_index("s") * NC + lax.axis_index("c")
        base = wid * RPW
        pltpu.sync_copy(tok_hbm.at[pl.ds(base, RPW)], stage_v)
        for r in range(RPW):
            for c in range(0, S - L + 1, L):
                flat_v[pl.ds(r * S + c, L)] = stage_v[r, pl.ds(c, L)]
            if tail % L:
                flat_v[pl.ds(r * S + tail, L)] = stage_v[r, pl.ds(tail, L)]
        pltpu.sync_copy(flat_v, out_hbm.at[pl.ds(base * S, RPW * S)])

    return depad(tokens)


def _soft_embedding_call(tokens_flat, wte_weight, learned_embedding, B, S, D, NT):
    NT8 = (NT // 8) * 8
    G = S - NT8
    GO = NT - NT8
    GR = S - NT
    info = plsc.get_sparse_core_info()
    NC, NS = info.num_cores, info.num_subcores
    NW = NC * NS
    RPW = B // NW
    C1 = min(G, 128)
    C2 = G - C1

    mesh = plsc.VectorSubcoreMesh(core_axis_name="c", subcore_axis_name="s")

    @functools.partial(
        pl.kernel,
        mesh=mesh,
        out_type=jax.ShapeDtypeStruct((B, S, D), jnp.float32),
        compiler_params=pltpu.CompilerParams(use_tc_tiling_on_sc=False),
        scratch_types=[
            pltpu.VMEM((RPW * S,), jnp.int32),
            pltpu.VMEM((_NBUF, G, D), jnp.float32),
            pltpu.VMEM((NT, D), jnp.float32),
            pltpu.SemaphoreType.DMA((_NBUF,)),
            pltpu.SemaphoreType.DMA((_NBUF,)),
        ],
    )
    def soft_embed(tok_hbm, wte_hbm, le_hbm, out_hbm, toks_v, gath_v, le_v,
                   gsem, wsem):
        wid = lax.axis_index("s") * NC + lax.axis_index("c")
        base = wid * RPW
        pltpu.sync_copy(tok_hbm.at[pl.ds(base * S, RPW * S)], toks_v)
        pltpu.sync_copy(le_hbm, le_v)

        def gather_copies(g, b):
            off = pl.multiple_of(g * S + NT8, 8)
            cs = [pltpu.make_async_copy(
                wte_hbm.at[toks_v.at[pl.ds(off, C1)]],
                gath_v.at[b, pl.ds(0, C1)], gsem.at[b])]
            if C2:
                cs.append(pltpu.make_async_copy(
                    wte_hbm.at[toks_v.at[pl.ds(off + C1, C2)]],
                    gath_v.at[b, pl.ds(C1, C2)], gsem.at[b]))
            return cs

        def write_copies(g, b):
            row = base + g
            return [
                pltpu.make_async_copy(
                    le_v, out_hbm.at[row, pl.ds(0, NT)], wsem.at[b]),
                pltpu.make_async_copy(
                    gath_v.at[b, pl.ds(GO, GR)],
                    out_hbm.at[row, pl.ds(NT, GR)], wsem.at[b]),
            ]

        def start(cs):
            for c in cs:
                c.start()

        def wait(cs):
            for c in cs:
                c.wait()

        for b in range(_NBUF):
            start(gather_copies(b, b))

        def outer(k, carry):
            g0 = k * _NBUF
            for b in range(_NBUF):
                g = g0 + b
                wait(gather_copies(g, b))
                start(write_copies(g, b))
                wait(write_copies(g, b))
                start(gather_copies(g + _NBUF, b))
            return carry

        lax.fori_loop(0, RPW // _NBUF - 1, outer, 0)

        for b in range(_NBUF):
            g = RPW - _NBUF + b
            wait(gather_copies(g, b))
            start(write_copies(g, b))
        for b in range(_NBUF):
            wait(write_copies(RPW - _NBUF + b, b))

    return soft_embed(tokens_flat, wte_weight, learned_embedding)


def kernel(tokens, wte_weight, learned_embedding):
    B, S = tokens.shape
    V, D = wte_weight.shape
    NT = learned_embedding.shape[0]
    tokens = tokens.astype(jnp.int32)
    tokens_flat = _depad_tokens_call(tokens, B, S)
    return _soft_embedding_call(
        tokens_flat, wte_weight, learned_embedding, B, S, D, NT)
